# baseline (speedup 1.0000x reference)
import functools

import jax
import jax.numpy as jnp
from jax import lax
from jax.experimental import pallas as pl
from jax.experimental.pallas import tpu as pltpu
from jax.experimental.pallas import tpu_sc as plsc

_CW = 128
_BN = 16384


def _make_sc_gather(num_rows, emb_dim, table_rows, chunks_per_worker, table_dtype):
    rows_per_worker = chunks_per_worker * _CW
    mesh = plsc.VectorSubcoreMesh(core_axis_name="c", subcore_axis_name="s")
    num_cores = plsc.get_sparse_core_info().num_cores

    @functools.partial(
        pl.kernel,
        mesh=mesh,
        out_type=jax.ShapeDtypeStruct((num_rows, emb_dim), table_dtype),
        scratch_types=[
            pltpu.VMEM((chunks_per_worker, _CW), jnp.int32),
            pltpu.VMEM((_CW, emb_dim), table_dtype),
            pltpu.SemaphoreType.DMA,
        ],
        compiler_params=pltpu.CompilerParams(use_tc_tiling_on_sc=False),
    )
    def gather_k(table_hbm, idx_hbm, out_hbm, idx_v, rows_v, sem):
        wid = lax.axis_index("s") * num_cores + lax.axis_index("c")
        pltpu.sync_copy(
            idx_hbm.at[pl.ds(wid * chunks_per_worker, chunks_per_worker)], idx_v
        )
        base_row = wid * rows_per_worker

        def body(j, carry):
            pltpu.async_copy(table_hbm.at[idx_v.at[j]], rows_v, sem).wait()
            pltpu.sync_copy(rows_v, out_hbm.at[pl.ds(base_row + j * _CW, _CW)])
            return carry

        lax.fori_loop(0, chunks_per_worker, body, 0)

    return gather_k


def _fuse_body(t_ref, w_ref, out_ref):
    dn = (((0,), (1,)), ((), ()))
    proj = lax.dot_general(
        t_ref[...], w_ref[...], dn, preferred_element_type=jnp.float32
    )
    out_ref[...] = jnp.concatenate(
        [proj[: _BN // 2], proj[_BN // 2 :]], axis=1
    )


def kernel(input, table, W):
    B, L = input.shape
    V, D = table.shape
    O = W.shape[0]
    n = B * L

    nblk = (V + _BN - 1) // _BN
    vpad = nblk * _BN

    idx = input.reshape(-1).astype(jnp.int32)
    j = idx % _BN
    ridx = (idx - j) + 2 * (j % (_BN // 2)) + j // (_BN // 2)
    ridx = ridx.reshape(n // _CW, _CW)

    tableT = table.T
    fused2 = pl.pallas_call(
        _fuse_body,
        grid=(nblk,),
        in_specs=[
            pl.BlockSpec((D, _BN), lambda i: (0, i)),
            pl.BlockSpec((O, D), lambda i: (0, 0)),
        ],
        out_specs=pl.BlockSpec((_BN // 2, 2 * O), lambda i: (i, 0)),
        out_shape=jax.ShapeDtypeStruct((vpad // 2, 2 * O), jnp.float32),
        compiler_params=pltpu.CompilerParams(
            dimension_semantics=("parallel",),
            fuse_transposed_lhs_in_matmul=True,
        ),
    )(tableT, W)
    fused_lin = fused2.reshape(vpad, O)

    chunks_per_worker = n // (_CW * 32)
    gather_k = _make_sc_gather(n, O, vpad, chunks_per_worker, fused_lin.dtype)
    out = gather_k(fused_lin, ridx)

    return out.reshape(B, L, O)


# speedup vs baseline: 1.4301x; 1.4301x over previous
import functools

import jax
import jax.numpy as jnp
from jax import lax
from jax.experimental import pallas as pl
from jax.experimental.pallas import tpu as pltpu
from jax.experimental.pallas import tpu_sc as plsc

_CW = 128
_BN = 16384


def _make_sc_gather(num_rows, emb_dim, chunks_per_worker, table_dtype):
    rows_per_worker = chunks_per_worker * _CW
    mesh = plsc.VectorSubcoreMesh(core_axis_name="c", subcore_axis_name="s")
    num_cores = plsc.get_sparse_core_info().num_cores

    @functools.partial(
        pl.kernel,
        mesh=mesh,
        out_type=jax.ShapeDtypeStruct((num_rows, emb_dim), table_dtype),
        scratch_types=[
            pltpu.VMEM((chunks_per_worker, _CW), jnp.int32),
            pltpu.VMEM((_CW, emb_dim), table_dtype),
            pltpu.SemaphoreType.DMA,
        ],
        compiler_params=pltpu.CompilerParams(use_tc_tiling_on_sc=False),
    )
    def gather_k(table_hbm, idx_hbm, out_hbm, idx_v, rows_v, sem):
        wid = lax.axis_index("s") * num_cores + lax.axis_index("c")
        pltpu.sync_copy(
            idx_hbm.at[pl.ds(wid * chunks_per_worker, chunks_per_worker)], idx_v
        )
        base_row = wid * rows_per_worker

        def body(j, carry):
            pltpu.async_copy(table_hbm.at[idx_v.at[j]], rows_v, sem).wait()
            pltpu.sync_copy(rows_v, out_hbm.at[pl.ds(base_row + j * _CW, _CW)])
            return carry

        lax.fori_loop(0, chunks_per_worker, body, 0)

    return gather_k


def _fuse_body(t_ref, w_ref, out_ref):
    dn = (((0,), (1,)), ((), ()))
    proj = lax.dot_general(
        t_ref[...], w_ref[...], dn, preferred_element_type=jnp.float32
    )
    out_ref[...] = jnp.concatenate(
        [proj[: _BN // 2], proj[_BN // 2 :]], axis=1
    )


def _xpose_body(in_ref, out_ref):
    half = in_ref.shape[1] // 2
    left = in_ref[:, :half]
    right = in_ref[:, half:]
    out_ref[...] = jnp.concatenate([left.T, right.T], axis=1)


def kernel(input, table, W):
    B, L = input.shape
    V, D = table.shape
    O = W.shape[0]
    n = B * L

    nblk = (V + _BN - 1) // _BN
    vpad = nblk * _BN

    q = jnp.arange(B // 2, dtype=jnp.int32)
    b_order = jnp.stack([q, q + B // 2], axis=1).reshape(-1)
    idx = input.T.astype(jnp.int32)[:, b_order].reshape(-1)

    j = idx % _BN
    ridx = (idx - j) + 2 * (j % (_BN // 2)) + j // (_BN // 2)
    ridx = ridx.reshape(n // _CW, _CW)

    tableT = table.T
    fused2 = pl.pallas_call(
        _fuse_body,
        grid=(nblk,),
        in_specs=[
            pl.BlockSpec((D, _BN), lambda i: (0, i)),
            pl.BlockSpec((O, D), lambda i: (0, 0)),
        ],
        out_specs=pl.BlockSpec((_BN // 2, 2 * O), lambda i: (i, 0)),
        out_shape=jax.ShapeDtypeStruct((vpad // 2, 2 * O), jnp.float32),
        compiler_params=pltpu.CompilerParams(
            dimension_semantics=("parallel",),
        ),
    )(tableT, W)
    fused_lin = fused2.reshape(vpad, O)

    chunks_per_worker = n // (_CW * 32)
    gather_k = _make_sc_gather(n, O, chunks_per_worker, fused_lin.dtype)
    out = gather_k(fused_lin, ridx)
    out_pairs = out.reshape(n // 2, 2 * O)

    xposed = pl.pallas_call(
        _xpose_body,
        grid=(L,),
        in_specs=[pl.BlockSpec((B // 2, 2 * O), lambda i: (i, 0))],
        out_specs=pl.BlockSpec((O, B), lambda i: (i, 0)),
        out_shape=jax.ShapeDtypeStruct((L * O, B), jnp.float32),
        compiler_params=pltpu.CompilerParams(
            dimension_semantics=("parallel",),
        ),
    )(out_pairs)

    return xposed.reshape(L, O, B).transpose(2, 0, 1)


# speedup vs baseline: 1.6995x; 1.1883x over previous
import functools

import jax
import jax.numpy as jnp
from jax import lax
from jax.experimental import pallas as pl
from jax.experimental.pallas import tpu as pltpu
from jax.experimental.pallas import tpu_sc as plsc

_CW = 128
_BN = 16384


def _make_sc_gather(num_rows, emb_dim, chunks_per_worker, table_dtype):
    rows_per_worker = chunks_per_worker * _CW
    mesh = plsc.VectorSubcoreMesh(core_axis_name="c", subcore_axis_name="s")
    num_cores = plsc.get_sparse_core_info().num_cores

    @functools.partial(
        pl.kernel,
        mesh=mesh,
        out_type=jax.ShapeDtypeStruct((num_rows, emb_dim), table_dtype),
        scratch_types=[
            pltpu.VMEM((chunks_per_worker, _CW), jnp.int32),
            pltpu.VMEM((2, _CW, emb_dim), table_dtype),
            pltpu.SemaphoreType.DMA,
            pltpu.SemaphoreType.DMA,
        ],
        compiler_params=pltpu.CompilerParams(use_tc_tiling_on_sc=False),
    )
    def gather_k(table_hbm, idx_hbm, out_hbm, idx_v, rows_v, gsem0, gsem1):
        wid = lax.axis_index("s") * num_cores + lax.axis_index("c")
        pltpu.sync_copy(
            idx_hbm.at[pl.ds(wid * chunks_per_worker, chunks_per_worker)], idx_v
        )
        base_row = wid * rows_per_worker
        gsems = (gsem0, gsem1)
        npairs = chunks_per_worker // 2

        def start_gather(j, buf):
            pltpu.async_copy(table_hbm.at[idx_v.at[j]], rows_v.at[buf], gsems[buf])

        def wait_gather(buf):
            pltpu.make_async_copy(
                table_hbm.at[idx_v.at[0]], rows_v.at[buf], gsems[buf]
            ).wait()

        start_gather(0, 0)

        def body(i, carry):
            ja = 2 * i
            start_gather(ja + 1, 1)
            wait_gather(0)
            pltpu.sync_copy(
                rows_v.at[0], out_hbm.at[pl.ds(base_row + ja * _CW, _CW)]
            )

            @pl.when(i < npairs - 1)
            def _():
                start_gather(ja + 2, 0)

            wait_gather(1)
            pltpu.sync_copy(
                rows_v.at[1], out_hbm.at[pl.ds(base_row + (ja + 1) * _CW, _CW)]
            )
            return carry

        lax.fori_loop(0, npairs, body, 0)

    return gather_k


def _fuse_body(t_ref, w_ref, out_ref):
    dn = (((0,), (1,)), ((), ()))
    proj = lax.dot_general(
        t_ref[...], w_ref[...], dn, preferred_element_type=jnp.float32
    )
    out_ref[...] = jnp.concatenate(
        [proj[: _BN // 2], proj[_BN // 2 :]], axis=1
    )


def _xpose_body(in_ref, out_ref):
    half = in_ref.shape[1] // 2
    left = in_ref[:, :half]
    right = in_ref[:, half:]
    out_ref[...] = jnp.concatenate([left.T, right.T], axis=1)


def kernel(input, table, W):
    B, L = input.shape
    V, D = table.shape
    O = W.shape[0]
    n = B * L

    nblk = (V + _BN - 1) // _BN
    vpad = nblk * _BN

    q = jnp.arange(B // 2, dtype=jnp.int32)
    b_order = jnp.stack([q, q + B // 2], axis=1).reshape(-1)
    idx = input.T.astype(jnp.int32)[:, b_order].reshape(-1)

    j = idx % _BN
    ridx = (idx - j) + 2 * (j % (_BN // 2)) + j // (_BN // 2)
    ridx = ridx.reshape(n // _CW, _CW)

    tableT = table.T
    fused2 = pl.pallas_call(
        _fuse_body,
        grid=(nblk,),
        in_specs=[
            pl.BlockSpec((D, _BN), lambda i: (0, i)),
            pl.BlockSpec((O, D), lambda i: (0, 0)),
        ],
        out_specs=pl.BlockSpec((_BN // 2, 2 * O), lambda i: (i, 0)),
        out_shape=jax.ShapeDtypeStruct((vpad // 2, 2 * O), jnp.float32),
        compiler_params=pltpu.CompilerParams(
            dimension_semantics=("parallel",),
        ),
    )(tableT, W)
    fused_lin = fused2.reshape(vpad, O)

    chunks_per_worker = n // (_CW * 32)
    gather_k = _make_sc_gather(n, O, chunks_per_worker, fused_lin.dtype)
    out = gather_k(fused_lin, ridx)
    out_pairs = out.reshape(n // 2, 2 * O)

    xposed = pl.pallas_call(
        _xpose_body,
        grid=(L,),
        in_specs=[pl.BlockSpec((B // 2, 2 * O), lambda i: (i, 0))],
        out_specs=pl.BlockSpec((O, B), lambda i: (i, 0)),
        out_shape=jax.ShapeDtypeStruct((L * O, B), jnp.float32),
        compiler_params=pltpu.CompilerParams(
            dimension_semantics=("parallel",),
        ),
    )(out_pairs)

    return xposed.reshape(L, O, B).transpose(2, 0, 1)
